# Initial kernel scaffold; baseline (speedup 1.0000x reference)
import jax
import jax.numpy as jnp
from jax.experimental import pallas as pl


def kernel(x, token_to_expert_indices, weights, gate_proj_weight, up_proj_weight, down_proj_weight):
    raise NotImplementedError("write your pallas kernel here")


# speedup vs baseline: 1.1628x; 1.1628x over previous
import functools

import jax
import jax.numpy as jnp
from jax.experimental import pallas as pl
from jax.experimental.pallas import tpu as pltpu


def _ffn_body(idx_ref, w_ref, x_ref, g_ref, u_ref, d_ref, out_ref):
    e = pl.program_id(0)
    xb = x_ref[...]
    g = jax.lax.dot_general(xb, g_ref[...], (((1,), (1,)), ((), ())),
                            preferred_element_type=jnp.float32)
    u = jax.lax.dot_general(xb, u_ref[...], (((1,), (1,)), ((), ())),
                            preferred_element_type=jnp.float32)
    h = g * jax.nn.sigmoid(g) * u
    idx = idx_ref[...]
    w = w_ref[...]
    we = jnp.sum(jnp.where(idx == e, w, 0.0), axis=1)
    h = h * we[:, None]
    y = jax.lax.dot_general(h.astype(jnp.bfloat16), d_ref[...],
                            (((1,), (1,)), ((), ())),
                            preferred_element_type=jnp.float32)
    t = pl.program_id(1)
    bt = y.shape[0]

    @pl.when(e == 0)
    def _init():
        out_ref[pl.ds(t * bt, bt), :] = y

    @pl.when(e != 0)
    def _acc():
        out_ref[pl.ds(t * bt, bt), :] += y


def kernel(x, token_to_expert_indices, weights, gate_proj_weight,
           up_proj_weight, down_proj_weight):
    T, DIM = x.shape
    E, INTER, _ = gate_proj_weight.shape
    TOPK = token_to_expert_indices.shape[1]
    BT = 256
    NT = T // BT

    xb = x.astype(jnp.bfloat16)
    gw = gate_proj_weight.astype(jnp.bfloat16)
    uw = up_proj_weight.astype(jnp.bfloat16)
    dw = down_proj_weight.astype(jnp.bfloat16)
    idx = token_to_expert_indices.astype(jnp.int32)

    out = pl.pallas_call(
        _ffn_body,
        grid=(E, NT),
        in_specs=[
            pl.BlockSpec((BT, TOPK), lambda e, t: (t, 0)),
            pl.BlockSpec((BT, TOPK), lambda e, t: (t, 0)),
            pl.BlockSpec((BT, DIM), lambda e, t: (t, 0)),
            pl.BlockSpec((None, INTER, DIM), lambda e, t: (e, 0, 0)),
            pl.BlockSpec((None, INTER, DIM), lambda e, t: (e, 0, 0)),
            pl.BlockSpec((None, DIM, INTER), lambda e, t: (e, 0, 0)),
        ],
        out_specs=pl.BlockSpec((T, DIM), lambda e, t: (0, 0)),
        out_shape=jax.ShapeDtypeStruct((T, DIM), jnp.float32),
        compiler_params=pltpu.CompilerParams(
            dimension_semantics=("arbitrary", "arbitrary"),
        ),
    )(idx, weights, xb, gw, uw, dw)
    return out
